# Initial kernel scaffold; baseline (speedup 1.0000x reference)
import jax
import jax.numpy as jnp
from jax.experimental import pallas as pl


def kernel(x, edge_index, mask_indices, W_enc, b_enc, W1, b1, W2, b2, mask_token):
    raise NotImplementedError("write your pallas kernel here")


# speedup vs baseline: 28.4802x; 28.4802x over previous
import functools

import jax
import jax.numpy as jnp
from jax import lax
from jax.experimental import pallas as pl
from jax.experimental.pallas import tpu as pltpu
from jax.experimental.pallas import tpu_sc as plsc

N = 10000
E = 320000
D = 128
H = 128
INJ = 2
NUM_MASK = 1500

NC = 2
NS = 16
NW = NC * NS
EPT = E // NW
K = 80
NCHUNK = EPT // K
KV = K // 16
MPAD = 1504
FLAGN = N + 16
NPAD = 10240
RPT = NPAD // NS
ZR = 128


def _sc_kernel_body(x_hbm, edge_hbm, mi_hbm, tok_hbm,
                    acc_out, deg_out, ism_out,
                    src_v, dst_v, flag_v, rows_v, mi_v, tok_v, ones_v, z640_v,
                    acc_sh, deg_sh, sem):
    c = lax.axis_index("c")
    s = lax.axis_index("s")
    wid = c * NS + s

    pltpu.sync_copy(mi_hbm, mi_v)
    pltpu.sync_copy(tok_hbm, tok_v)

    zero16f = jnp.zeros((16,), jnp.float32)
    ones16f = jnp.ones((16,), jnp.float32)
    zeros16i = jnp.zeros((16,), jnp.int32)
    ones16i = jnp.ones((16,), jnp.int32)

    def _zf(i, _):
        flag_v[pl.ds(i * 16, 16)] = zero16f
        return 0
    lax.fori_loop(0, FLAGN // 16, _zf, 0)

    def _zr(i, _):
        rows_v[i // (D // 16), pl.ds((i % (D // 16)) * 16, 16)] = zero16f
        return 0
    lax.fori_loop(0, K * (D // 16), _zr, 0)

    def _zz(i, _):
        z640_v[pl.ds(i * 16, 16)] = zero16f
        ones_v[pl.ds(i * 16, 16)] = ones16f
        return 0
    lax.fori_loop(0, RPT // 16, _zz, 0)

    def _bf(i, _):
        mv = mi_v[pl.ds(i * 16, 16)]
        plsc.store_scatter(flag_v, [mv], ones16f)
        return 0
    lax.fori_loop(0, MPAD // 16, _bf, 0)

    for z in range(RPT // K):
        pltpu.sync_copy(rows_v, acc_sh.at[pl.ds(s * RPT + z * K, K)])
    pltpu.sync_copy(z640_v, deg_sh.at[pl.ds(s * RPT, RPT)])
    plsc.subcore_barrier()

    t0 = tok_v[0, pl.ds(0, 16)]
    t1 = tok_v[1, pl.ds(0, 16)]
    iota16 = lax.iota(jnp.int32, 16)

    def _chunk(j, _):
        pltpu.sync_copy(edge_hbm.at[0, wid, j], src_v)
        pltpu.sync_copy(edge_hbm.at[1, wid, j], dst_v)
        pltpu.async_copy(x_hbm.at[src_v.at[0]], rows_v, sem).wait()
        for jj in range(KV):
            sv = src_v[0, pl.ds(jj * 16, 16)]
            fl = plsc.load_gather(flag_v, [sv])
            m = fl > 0.5
            rowpos = iota16 + jj * 16
            plsc.store_scatter(rows_v, [rowpos, zeros16i], t0, mask=m)
            plsc.store_scatter(rows_v, [rowpos, ones16i], t1, mask=m)
        pltpu.sync_copy(rows_v, acc_sh.at[dst_v.at[0]], add=True)
        pltpu.sync_copy(ones_v.at[pl.ds(0, K)], deg_sh.at[dst_v.at[0]],
                        add=True)
        return 0

    lax.fori_loop(0, NCHUNK, _chunk, 0)

    plsc.subcore_barrier()
    pltpu.sync_copy(acc_sh.at[pl.ds(s * RPT, RPT)],
                    acc_out.at[c, pl.ds(s * RPT, RPT)])
    pltpu.sync_copy(deg_sh.at[pl.ds(s * RPT, RPT)], deg_out.at[c, s])

    @pl.when(jnp.logical_and(c == 0, s == 0))
    def _():
        pltpu.sync_copy(flag_v.at[pl.ds(0, N)], ism_out)


_sc_kernel = functools.partial(
    pl.kernel,
    out_type=[
        jax.ShapeDtypeStruct((NC, NPAD, D), jnp.float32),
        jax.ShapeDtypeStruct((NC, NS, RPT), jnp.float32),
        jax.ShapeDtypeStruct((N,), jnp.float32),
    ],
    mesh=plsc.VectorSubcoreMesh(core_axis_name="c", subcore_axis_name="s"),
    scratch_types=[
        pltpu.VMEM((1, K), jnp.int32),
        pltpu.VMEM((1, K), jnp.int32),
        pltpu.VMEM((FLAGN,), jnp.float32),
        pltpu.VMEM((K, D), jnp.float32),
        pltpu.VMEM((MPAD,), jnp.int32),
        pltpu.VMEM((2, 16), jnp.float32),
        pltpu.VMEM((RPT,), jnp.float32),
        pltpu.VMEM((RPT,), jnp.float32),
        pltpu.VMEM_SHARED((NPAD, D), jnp.float32),
        pltpu.VMEM_SHARED((NPAD,), jnp.float32),
        pltpu.SemaphoreType.DMA,
    ],
    compiler_params=pltpu.CompilerParams(needs_layout_passes=False),
)(_sc_kernel_body)


CHUNK_TC = 1000


def _tc_kernel_body(acc_ref, degt_ref, ism_ref, x_ref, mi_ref, tok_ref,
                    wenc_ref, benc_ref, w1_ref, b1_ref, w2_ref, b2_ref,
                    o_ref):
    acc = acc_ref[0, :N] + acc_ref[1, :N]
    degt = degt_ref[...]
    deg = degt[:N, 0:1] + degt[:N, 1:2]
    agg = acc / jnp.maximum(deg, 1.0)
    ism = ism_ref[...]
    x = x_ref[...]
    tok = tok_ref[...]
    h = agg + x
    h01 = h[:, :INJ] + (tok - x[:, :INJ]) * ism
    h = jnp.concatenate([h01, h[:, INJ:]], axis=1)
    emb = jnp.maximum(
        jnp.dot(h, wenc_ref[...], preferred_element_type=jnp.float32)
        + benc_ref[...], 0.0)

    mi = mi_ref[...]
    embm = jnp.zeros((NUM_MASK, H), jnp.float32)
    orig = jnp.zeros((NUM_MASK, INJ), jnp.float32)
    for ci in range(N // CHUNK_TC):
        cols = lax.broadcasted_iota(jnp.int32, (1, CHUNK_TC), 1) + ci * CHUNK_TC
        oh = (mi == cols).astype(jnp.float32)
        embm = embm + jnp.dot(oh, emb[ci * CHUNK_TC:(ci + 1) * CHUNK_TC],
                              preferred_element_type=jnp.float32)
        orig = orig + jnp.dot(oh, x[ci * CHUNK_TC:(ci + 1) * CHUNK_TC, :INJ],
                              preferred_element_type=jnp.float32)

    hid = jnp.maximum(
        jnp.dot(embm, w1_ref[...], preferred_element_type=jnp.float32)
        + b1_ref[...], 0.0)
    pred = jnp.dot(hid, w2_ref[...], preferred_element_type=jnp.float32) \
        + b2_ref[...]
    loss = jnp.sum(jnp.abs(pred - orig)) / (NUM_MASK * INJ)
    o_ref[...] = jnp.reshape(loss, (1, 1))


def kernel(x, edge_index, mask_indices, W_enc, b_enc, W1, b1, W2, b2,
           mask_token):
    edge3 = edge_index.reshape(2, NW, NCHUNK, 1, K)
    mi_pad = jnp.concatenate(
        [mask_indices, jnp.full((MPAD - NUM_MASK,), N, jnp.int32)])
    tokb = jnp.broadcast_to(mask_token[:, None], (INJ, 16))

    acc_p, deg_p, ism = _sc_kernel(x, edge3, mi_pad, tokb)

    loss2 = pl.pallas_call(
        _tc_kernel_body,
        out_shape=jax.ShapeDtypeStruct((1, 1), jnp.float32),
    )(acc_p, deg_p.reshape(NC, NPAD).T, ism[:, None], x, mask_indices[:, None],
      mask_token[None, :], W_enc, b_enc[None, :], W1, b1[None, :],
      W2, b2[None, :])
    return loss2[0, 0]


# speedup vs baseline: 88.9919x; 3.1247x over previous
import functools

import jax
import jax.numpy as jnp
from jax import lax
from jax.experimental import pallas as pl
from jax.experimental.pallas import tpu as pltpu
from jax.experimental.pallas import tpu_sc as plsc

N = 10000
E = 320000
D = 128
H = 128
INJ = 2
NUM_MASK = 1500

NC = 2
NS = 16
NW = NC * NS
EPT = E // NW
K = 80
KV = K // 16
MPAD = 1504
MG = 1520
NGCH = MG // K
FLAGN = N + 16
ACC_R = 2048
DUMP = 1536
PEND = EPT + 96
SPT = ACC_R // NS


def _sc_kernel_body(x_hbm, edge_hbm, mi_hbm, mig_hbm, tok_hbm,
                    acc_out, deg_out, w_out, xrows_out,
                    src_v, dst_v, psrc_v, pslot_v, slot_v, rows_v, mi_v,
                    tok_v, srcw_v, slotw_v, migw_v, ones_v, z128_v, w_v,
                    acc_sh, deg_sh, sem):
    c = lax.axis_index("c")
    s = lax.axis_index("s")
    wid = c * NS + s

    pltpu.sync_copy(edge_hbm.at[0, wid], src_v)
    pltpu.sync_copy(edge_hbm.at[1, wid], dst_v)
    pltpu.sync_copy(mi_hbm, mi_v)
    pltpu.sync_copy(tok_hbm, tok_v)

    zero16f = jnp.zeros((16,), jnp.float32)
    ones16f = jnp.ones((16,), jnp.float32)
    zeros16i = jnp.zeros((16,), jnp.int32)
    ones16i = jnp.ones((16,), jnp.int32)
    neg16i = jnp.full((16,), -1, jnp.int32)
    dump16i = jnp.full((16,), DUMP, jnp.int32)
    iota16 = lax.iota(jnp.int32, 16)

    def _zf(i, _):
        slot_v[pl.ds(i * 16, 16)] = neg16i
        return 0
    lax.fori_loop(0, FLAGN // 16, _zf, 0)

    def _zr(i, _):
        rows_v[i // (D // 16), pl.ds((i % (D // 16)) * 16, 16)] = zero16f
        return 0
    lax.fori_loop(0, K * (D // 16), _zr, 0)

    def _zz(i, _):
        z128_v[pl.ds(i * 16, 16)] = zero16f
        return 0
    lax.fori_loop(0, SPT // 16, _zz, 0)
    for jj in range(KV):
        ones_v[pl.ds(jj * 16, 16)] = ones16f

    def _bf(i, _):
        mv = mi_v[pl.ds(i * 16, 16)]
        plsc.store_scatter(slot_v, [mv], iota16 + i * 16)
        return 0
    lax.fori_loop(0, MPAD // 16, _bf, 0)

    pltpu.sync_copy(rows_v, acc_sh.at[pl.ds(s * SPT, K)])
    pltpu.sync_copy(rows_v.at[pl.ds(0, SPT - K)],
                    acc_sh.at[pl.ds(s * SPT + K, SPT - K)])
    pltpu.sync_copy(z128_v, deg_sh.at[pl.ds(s * SPT, SPT)])
    plsc.subcore_barrier()

    t0 = tok_v[0, pl.ds(0, 16)]
    t1 = tok_v[1, pl.ds(0, 16)]

    @pl.when(wid < NGCH)
    def _():
        pltpu.sync_copy(mig_hbm.at[wid], migw_v)
        pltpu.async_copy(x_hbm.at[migw_v.at[0]], rows_v, sem).wait()
        pltpu.sync_copy(rows_v, xrows_out.at[pl.ds(wid * K, K)])

    @pl.when(jnp.logical_and(c == 0, s == 0))
    def _():
        def _zw(i, _):
            w_v[pl.ds(i * 16, 16)] = zero16f
            return 0
        lax.fori_loop(0, ACC_R // 16, _zw, 0)

        def _bw(i, _):
            mv = mi_v[pl.ds(i * 16, 16)]
            rep = plsc.load_gather(slot_v, [mv])
            rep = jnp.where(rep < 0, dump16i, rep)
            plsc.addupdate_scatter(w_v, [rep], ones16f)
            return 0
        lax.fori_loop(0, MPAD // 16, _bw, 0)
        pltpu.sync_copy(w_v, w_out)

    def _scan(i, cnt):
        dv = dst_v[pl.ds(i * 16, 16)]
        sl = plsc.load_gather(slot_v, [dv])
        m = sl >= 0
        cum = lax.cumsum(jnp.where(m, 1, 0), axis=0)
        pos = cnt + cum - 1
        sv = src_v[pl.ds(i * 16, 16)]
        plsc.store_scatter(psrc_v, [pos], sv, mask=m)
        plsc.store_scatter(pslot_v, [pos], sl, mask=m)
        return cnt + jnp.max(cum)

    cnt = lax.fori_loop(0, EPT // 16, _scan, jnp.int32(0))

    for jj in range(KV):
        plsc.store_scatter(psrc_v, [cnt + iota16 + jj * 16], zeros16i)
        plsc.store_scatter(pslot_v, [cnt + iota16 + jj * 16], dump16i)

    def _chunk(g, _):
        base = g * K
        svs = []
        for jj in range(KV):
            sv = psrc_v[pl.ds(base + jj * 16, 16)]
            slv = pslot_v[pl.ds(base + jj * 16, 16)]
            srcw_v[0, pl.ds(jj * 16, 16)] = sv
            slotw_v[0, pl.ds(jj * 16, 16)] = slv
            svs.append(sv)
        pltpu.async_copy(x_hbm.at[srcw_v.at[0]], rows_v, sem).wait()
        for jj in range(KV):
            fl = plsc.load_gather(slot_v, [svs[jj]])
            m = fl >= 0
            rowpos = iota16 + jj * 16
            plsc.store_scatter(rows_v, [rowpos, zeros16i], t0, mask=m)
            plsc.store_scatter(rows_v, [rowpos, ones16i], t1, mask=m)
        pltpu.sync_copy(rows_v, acc_sh.at[slotw_v.at[0]], add=True)
        pltpu.sync_copy(ones_v.at[pl.ds(0, K)], deg_sh.at[slotw_v.at[0]],
                        add=True)
        return 0

    lax.fori_loop(0, (cnt + K - 1) // K, _chunk, 0)

    plsc.subcore_barrier()
    pltpu.sync_copy(acc_sh.at[pl.ds(s * SPT, SPT)],
                    acc_out.at[c, pl.ds(s * SPT, SPT)])
    pltpu.sync_copy(deg_sh.at[pl.ds(s * SPT, SPT)], deg_out.at[c, s])


_sc_kernel = functools.partial(
    pl.kernel,
    out_type=[
        jax.ShapeDtypeStruct((NC, ACC_R, D), jnp.float32),
        jax.ShapeDtypeStruct((NC, NS, SPT), jnp.float32),
        jax.ShapeDtypeStruct((ACC_R,), jnp.float32),
        jax.ShapeDtypeStruct((MG, D), jnp.float32),
    ],
    mesh=plsc.VectorSubcoreMesh(core_axis_name="c", subcore_axis_name="s"),
    scratch_types=[
        pltpu.VMEM((EPT,), jnp.int32),
        pltpu.VMEM((EPT,), jnp.int32),
        pltpu.VMEM((PEND,), jnp.int32),
        pltpu.VMEM((PEND,), jnp.int32),
        pltpu.VMEM((FLAGN,), jnp.int32),
        pltpu.VMEM((K, D), jnp.float32),
        pltpu.VMEM((MPAD,), jnp.int32),
        pltpu.VMEM((2, 16), jnp.float32),
        pltpu.VMEM((1, K), jnp.int32),
        pltpu.VMEM((1, K), jnp.int32),
        pltpu.VMEM((1, K), jnp.int32),
        pltpu.VMEM((K,), jnp.float32),
        pltpu.VMEM((SPT,), jnp.float32),
        pltpu.VMEM((ACC_R,), jnp.float32),
        pltpu.VMEM_SHARED((ACC_R, D), jnp.float32),
        pltpu.VMEM_SHARED((ACC_R,), jnp.float32),
        pltpu.SemaphoreType.DMA,
    ],
    compiler_params=pltpu.CompilerParams(needs_layout_passes=False),
)(_sc_kernel_body)


def _tc_kernel_body(acc_ref, degt_ref, w_ref, xr_ref, tok_ref,
                    wenc_ref, benc_ref, w1_ref, b1_ref, w2_ref, b2_ref,
                    o_ref):
    acc = acc_ref[0, :MPAD] + acc_ref[1, :MPAD]
    degt = degt_ref[...]
    deg = degt[:MPAD, 0:1] + degt[:MPAD, 1:2]
    agg = acc / jnp.maximum(deg, 1.0)
    xr = xr_ref[...][:MPAD]
    tok = tok_ref[...]
    xm01 = jnp.broadcast_to(tok, (MPAD, INJ))
    h = agg + jnp.concatenate([xm01, xr[:, INJ:]], axis=1)
    emb = jnp.maximum(
        jnp.dot(h, wenc_ref[...], preferred_element_type=jnp.float32)
        + benc_ref[...], 0.0)
    hid = jnp.maximum(
        jnp.dot(emb, w1_ref[...], preferred_element_type=jnp.float32)
        + b1_ref[...], 0.0)
    pred = jnp.dot(hid, w2_ref[...], preferred_element_type=jnp.float32) \
        + b2_ref[...]
    wv = w_ref[...][:MPAD]
    loss = jnp.sum(jnp.abs(pred - xr[:, :INJ]) * wv) / (NUM_MASK * INJ)
    o_ref[...] = jnp.reshape(loss, (1, 1))


def kernel(x, edge_index, mask_indices, W_enc, b_enc, W1, b1, W2, b2,
           mask_token):
    edge3 = edge_index.reshape(2, NW, EPT)
    mi_pad = jnp.concatenate(
        [mask_indices, jnp.full((MPAD - NUM_MASK,), N, jnp.int32)])
    mi_g = jnp.concatenate(
        [mask_indices, jnp.zeros((MG - NUM_MASK,), jnp.int32)])
    mi_g3 = mi_g.reshape(NGCH, 1, K)
    tokb = jnp.broadcast_to(mask_token[:, None], (INJ, 16))

    acc_p, deg_p, w, xrows = _sc_kernel(x, edge3, mi_pad, mi_g3, tokb)

    loss2 = pl.pallas_call(
        _tc_kernel_body,
        out_shape=jax.ShapeDtypeStruct((1, 1), jnp.float32),
    )(acc_p, deg_p.reshape(NC, ACC_R).T, w[:, None], xrows,
      mask_token[None, :], W_enc, b_enc[None, :], W1, b1[None, :],
      W2, b2[None, :])
    return loss2[0, 0]
